# baseline (speedup 1.0000x reference)
import functools

import jax
import jax.numpy as jnp
from jax import lax
from jax.experimental import pallas as pl
from jax.experimental.pallas import tpu as pltpu
from jax.experimental.pallas import tpu_sc as plsc

_D = 64
_DO = 128
_NC, _NS = 2, 16
_NW = _NC * _NS
_CHUNK = 320
_NBUF = 4
_P = 2

_MESH = plsc.VectorSubcoreMesh(
    core_axis_name="c", subcore_axis_name="s",
    num_cores=_NC, num_subcores=_NS)


@jax.jit
def _gather(idx_flat, table):
    total = idx_flat.shape[0]
    per_w = total // _NW
    nchunk = per_w // _CHUNK
    assert per_w % _CHUNK == 0 and nchunk % _NBUF == 0


    @functools.partial(
        pl.kernel,
        out_type=jax.ShapeDtypeStruct((total, _DO), jnp.float32),
        mesh=_MESH,
        scratch_types=(
            [pltpu.VMEM((per_w,), jnp.int32)]
            + [pltpu.VMEM((_CHUNK, _D), jnp.float32) for _ in range(_NBUF)]
            + [pltpu.SemaphoreType.DMA for _ in range(2 * _NBUF)]
        ),
        compiler_params=pltpu.CompilerParams(use_tc_tiling_on_sc=False),
    )
    def k(idx_hbm, table_hbm, out_hbm, idx_all, *rest):
        bufs = rest[:_NBUF]
        gsem = rest[_NBUF:2 * _NBUF]
        wsem = rest[2 * _NBUF:]

        wid = lax.axis_index("s") * _NC + lax.axis_index("c")
        base = wid * per_w
        pltpu.sync_copy(idx_hbm.at[pl.ds(base, per_w)], idx_all)

        def idx_slc(c):
            return idx_all.at[pl.ds(c * _CHUNK, _CHUNK)]

        def out_slc(c):
            return out_hbm.at[pl.ds(base + c * _CHUNK, _CHUNK), pl.ds(0, _D)]

        def start_gather(c, b):
            pltpu.async_copy(table_hbm.at[idx_slc(c)], bufs[b], gsem[b])

        def wait_gather(c, b):
            pltpu.make_async_copy(
                table_hbm.at[idx_slc(c)], bufs[b], gsem[b]).wait()

        def start_wb(c, b):
            pltpu.async_copy(bufs[b], out_slc(c), wsem[b])

        def wait_wb(c, b):
            pltpu.make_async_copy(bufs[b], out_slc(c), wsem[b]).wait()

        for c0 in range(_P):
            start_gather(c0, c0 % _NBUF)

        @pl.loop(0, nchunk, step=_NBUF)
        def _(g):
            for b in range(_NBUF):
                c = g + b
                bb = (b + _P) % _NBUF

                @pl.when(c + _P < nchunk)
                def _():
                    @pl.when(c + _P >= _NBUF)
                    def _():
                        wait_wb(c + _P - _NBUF, bb)
                    start_gather(c + _P, bb)

                wait_gather(c, b)
                start_wb(c, b)

        for t in range(_NBUF):
            wait_wb(nchunk - _NBUF + t, t)

    return k(idx_flat, table)


def kernel(pos_indicies, embed_weight):
    idx = pos_indicies.astype(jnp.int32).reshape(-1)
    d = embed_weight.shape[-1]
    out = _gather(idx, embed_weight)
    return out[:, :d].reshape(pos_indicies.shape + (d,))


# speedup vs baseline: 1.0014x; 1.0014x over previous
import functools

import jax
import jax.numpy as jnp
from jax import lax
from jax.experimental import pallas as pl
from jax.experimental.pallas import tpu as pltpu
from jax.experimental.pallas import tpu_sc as plsc

_D = 64
_DO = 128
_NC, _NS = 2, 16
_NW = _NC * _NS
_CHUNK = 400
_NBUF = 4
_P = 2

_MESH = plsc.VectorSubcoreMesh(
    core_axis_name="c", subcore_axis_name="s",
    num_cores=_NC, num_subcores=_NS)


@jax.jit
def _gather(idx_flat, table):
    total = idx_flat.shape[0]
    per_w = total // _NW
    nchunk = per_w // _CHUNK
    assert per_w % _CHUNK == 0 and nchunk % _NBUF == 0


    @functools.partial(
        pl.kernel,
        out_type=jax.ShapeDtypeStruct((total, _DO), jnp.float32),
        mesh=_MESH,
        scratch_types=(
            [pltpu.VMEM((per_w,), jnp.int32)]
            + [pltpu.VMEM((_CHUNK, _D), jnp.float32) for _ in range(_NBUF)]
            + [pltpu.SemaphoreType.DMA for _ in range(2 * _NBUF)]
        ),
        compiler_params=pltpu.CompilerParams(use_tc_tiling_on_sc=False),
    )
    def k(idx_hbm, table_hbm, out_hbm, idx_all, *rest):
        bufs = rest[:_NBUF]
        gsem = rest[_NBUF:2 * _NBUF]
        wsem = rest[2 * _NBUF:]

        wid = lax.axis_index("s") * _NC + lax.axis_index("c")
        base = wid * per_w
        pltpu.sync_copy(idx_hbm.at[pl.ds(base, per_w)], idx_all)

        def idx_slc(c):
            return idx_all.at[pl.ds(c * _CHUNK, _CHUNK)]

        def out_slc(c):
            return out_hbm.at[pl.ds(base + c * _CHUNK, _CHUNK), pl.ds(0, _D)]

        def start_gather(c, b):
            pltpu.async_copy(table_hbm.at[idx_slc(c)], bufs[b], gsem[b])

        def wait_gather(c, b):
            pltpu.make_async_copy(
                table_hbm.at[idx_slc(c)], bufs[b], gsem[b]).wait()

        def start_wb(c, b):
            pltpu.async_copy(bufs[b], out_slc(c), wsem[b])

        def wait_wb(c, b):
            pltpu.make_async_copy(bufs[b], out_slc(c), wsem[b]).wait()

        for c0 in range(_P):
            start_gather(c0, c0 % _NBUF)

        @pl.loop(0, nchunk, step=_NBUF)
        def _(g):
            for b in range(_NBUF):
                c = g + b
                bb = (b + _P) % _NBUF

                @pl.when(c + _P < nchunk)
                def _():
                    @pl.when(c + _P >= _NBUF)
                    def _():
                        wait_wb(c + _P - _NBUF, bb)
                    start_gather(c + _P, bb)

                wait_gather(c, b)
                start_wb(c, b)

        for t in range(_NBUF):
            wait_wb(nchunk - _NBUF + t, t)

    return k(idx_flat, table)


def kernel(pos_indicies, embed_weight):
    idx = pos_indicies.astype(jnp.int32).reshape(-1)
    d = embed_weight.shape[-1]
    out = _gather(idx, embed_weight)
    return out[:, :d].reshape(pos_indicies.shape + (d,))
